# baseline (speedup 1.0000x reference)
import jax
import jax.numpy as jnp
from jax.experimental import pallas as pl

_DARK = 0.1
_EPS = 1e-08
_QS = (0.25, 0.75)


def _stage1_body(r_ref, w_ref, myr, mxr, myw, mxw):
    for src, my, mx in ((r_ref, myr, mxr), (w_ref, myw, mxw)):
        z = jnp.maximum(src[0] - _DARK, 0.0)
        my[0] = jnp.sum(z, axis=1, keepdims=True)
        mx[0] = jnp.sum(z, axis=0, keepdims=True)


def _stage2_body(myr, mxr, myw, mxw, out, *, B, C):
    BC, H = myr.shape
    f32 = jnp.float32
    ik = jax.lax.broadcasted_iota(jnp.int32, (H, H), 0)
    ij = jax.lax.broadcasted_iota(jnp.int32, (H, H), 1)
    L = (ik <= ij).astype(f32)

    def cum(ref):
        return jax.lax.dot(ref[...], L, precision=jax.lax.Precision.HIGHEST)

    cyr, cxr, cyw, cxw = cum(myr), cum(mxr), cum(myw), cum(mxw)

    gi = jax.lax.broadcasted_iota(jnp.int32, (B, BC), 1)
    gb = jax.lax.broadcasted_iota(jnp.int32, (B, BC), 0)
    G_bc = (gi // C == gb).astype(f32)
    si = jax.lax.broadcasted_iota(jnp.int32, (BC, B), 0)
    sb = jax.lax.broadcasted_iota(jnp.int32, (BC, B), 1)
    S_cb = (si // C == sb).astype(f32)

    def mm(a, b):
        return jax.lax.dot(a, b, precision=jax.lax.Precision.HIGHEST)

    tot_r = mm(G_bc, jnp.sum(myr[...], axis=1, keepdims=True)) + _EPS
    tot_w = mm(G_bc, jnp.sum(myw[...], axis=1, keepdims=True)) + _EPS

    scale = 20.0 / f32(H)
    loss = jnp.zeros((1, 1), f32)
    ones_b = jnp.ones((1, B), f32)
    for q in _QS:
        tr = mm(S_cb, q * tot_r)
        tw = mm(S_cb, q * tot_w)

        def count(cm, t):
            c = jnp.sum((cm < t).astype(f32), axis=1, keepdims=True)
            return c

        dqy = mm(G_bc, count(cyr, tr) - count(cyw, tw)) / C
        dqx = mm(G_bc, count(cxr, tr) - count(cxw, tw)) / C
        d4 = (dqy * scale) ** 4 + (dqx * scale) ** 4
        loss = loss + mm(ones_b, d4) / (2 * B)
    out[...] = loss


def kernel(ref_image, warped_image):
    import functools
    B, C, H, W = ref_image.shape
    BC = B * C
    r3 = ref_image.reshape(BC, H, W)
    w3 = warped_image.reshape(BC, H, W)

    my_shape = jax.ShapeDtypeStruct((BC, H, 1), jnp.float32)
    mx_shape = jax.ShapeDtypeStruct((BC, 1, W), jnp.float32)
    myr, mxr, myw, mxw = pl.pallas_call(
        _stage1_body,
        grid=(BC,),
        in_specs=[
            pl.BlockSpec((1, H, W), lambda i: (i, 0, 0)),
            pl.BlockSpec((1, H, W), lambda i: (i, 0, 0)),
        ],
        out_specs=[
            pl.BlockSpec((1, H, 1), lambda i: (i, 0, 0)),
            pl.BlockSpec((1, 1, W), lambda i: (i, 0, 0)),
            pl.BlockSpec((1, H, 1), lambda i: (i, 0, 0)),
            pl.BlockSpec((1, 1, W), lambda i: (i, 0, 0)),
        ],
        out_shape=[my_shape, mx_shape, my_shape, mx_shape],
    )(r3, w3)

    loss = pl.pallas_call(
        functools.partial(_stage2_body, B=B, C=C),
        out_shape=jax.ShapeDtypeStruct((1, 1), jnp.float32),
    )(myr.reshape(BC, H), mxr.reshape(BC, W),
      myw.reshape(BC, H), mxw.reshape(BC, W))
    return loss.reshape(())


# speedup vs baseline: 1.2534x; 1.2534x over previous
import jax
import jax.numpy as jnp
from jax.experimental import pallas as pl

_DARK = 0.1
_EPS = 1e-08
_QS = (0.25, 0.75)


def _stage1_body(r_ref, w_ref, myr, mxr, myw, mxw):
    for src, my, mx in ((r_ref, myr, mxr), (w_ref, myw, mxw)):
        z = jnp.maximum(src[...] - _DARK, 0.0)
        my[...] = jnp.sum(z, axis=2, keepdims=True)
        mx[...] = jnp.sum(z, axis=1, keepdims=True)


def _stage2_body(myr, mxr, myw, mxw, out, *, B, C):
    BC, H = myr.shape
    f32 = jnp.float32
    ik = jax.lax.broadcasted_iota(jnp.int32, (H, H), 0)
    ij = jax.lax.broadcasted_iota(jnp.int32, (H, H), 1)
    L = (ik <= ij).astype(f32)

    def cum(ref):
        return jax.lax.dot(ref[...], L, precision=jax.lax.Precision.HIGHEST)

    cyr, cxr, cyw, cxw = cum(myr), cum(mxr), cum(myw), cum(mxw)

    gi = jax.lax.broadcasted_iota(jnp.int32, (B, BC), 1)
    gb = jax.lax.broadcasted_iota(jnp.int32, (B, BC), 0)
    G_bc = (gi // C == gb).astype(f32)
    si = jax.lax.broadcasted_iota(jnp.int32, (BC, B), 0)
    sb = jax.lax.broadcasted_iota(jnp.int32, (BC, B), 1)
    S_cb = (si // C == sb).astype(f32)

    def mm(a, b):
        return jax.lax.dot(a, b, precision=jax.lax.Precision.HIGHEST)

    tot_r = mm(G_bc, jnp.sum(myr[...], axis=1, keepdims=True)) + _EPS
    tot_w = mm(G_bc, jnp.sum(myw[...], axis=1, keepdims=True)) + _EPS

    scale = 20.0 / f32(H)
    loss = jnp.zeros((1, 1), f32)
    ones_b = jnp.ones((1, B), f32)
    for q in _QS:
        tr = mm(S_cb, q * tot_r)
        tw = mm(S_cb, q * tot_w)

        def count(cm, t):
            c = jnp.sum((cm < t).astype(f32), axis=1, keepdims=True)
            return c

        dqy = mm(G_bc, count(cyr, tr) - count(cyw, tw)) / C
        dqx = mm(G_bc, count(cxr, tr) - count(cxw, tw)) / C
        d4 = (dqy * scale) ** 4 + (dqx * scale) ** 4
        loss = loss + mm(ones_b, d4) / (2 * B)
    out[...] = loss


def kernel(ref_image, warped_image):
    import functools
    B, C, H, W = ref_image.shape
    BC = B * C
    r3 = ref_image.reshape(BC, H, W)
    w3 = warped_image.reshape(BC, H, W)

    my_shape = jax.ShapeDtypeStruct((BC, H, 1), jnp.float32)
    mx_shape = jax.ShapeDtypeStruct((BC, 1, W), jnp.float32)
    IM = 4
    myr, mxr, myw, mxw = pl.pallas_call(
        _stage1_body,
        grid=(BC // IM,),
        in_specs=[
            pl.BlockSpec((IM, H, W), lambda i: (i, 0, 0)),
            pl.BlockSpec((IM, H, W), lambda i: (i, 0, 0)),
        ],
        out_specs=[
            pl.BlockSpec((IM, H, 1), lambda i: (i, 0, 0)),
            pl.BlockSpec((IM, 1, W), lambda i: (i, 0, 0)),
            pl.BlockSpec((IM, H, 1), lambda i: (i, 0, 0)),
            pl.BlockSpec((IM, 1, W), lambda i: (i, 0, 0)),
        ],
        out_shape=[my_shape, mx_shape, my_shape, mx_shape],
    )(r3, w3)

    loss = pl.pallas_call(
        functools.partial(_stage2_body, B=B, C=C),
        out_shape=jax.ShapeDtypeStruct((1, 1), jnp.float32),
    )(myr.reshape(BC, H), mxr.reshape(BC, W),
      myw.reshape(BC, H), mxw.reshape(BC, W))
    return loss.reshape(())
